# baseline (speedup 1.0000x reference)
import functools

import jax
import jax.numpy as jnp
from jax.experimental import pallas as pl
import jax.experimental.pallas.tpu as pltpu

B = 4096
D = 128
BM = 1024
BN = 256
GRID = B // BM


def _loss_kernel(q_ref, t_ref, out_ref, ut_ref):
    i = pl.program_id(0)

    @pl.when(i == 0)
    def _init():
        t = t_ref[...]
        ut_ref[...] = t / jnp.sqrt(jnp.sum(t * t, axis=1, keepdims=True))
        out_ref[...] = jnp.zeros((1, 1), jnp.float32)

    utb = ut_ref[pl.ds(i * BM, BM), :]
    q = q_ref[...]
    uq = q / jnp.sqrt(jnp.sum(q * q, axis=1, keepdims=True))
    both = jnp.concatenate((utb, uq), axis=0)

    rows = jax.lax.broadcasted_iota(jnp.int32, (BM, BN), 0)
    cols = jax.lax.broadcasted_iota(jnp.int32, (BM, BN), 1)

    m = jnp.full((BM, 1), -4.0, jnp.float32)
    pick = jnp.zeros((BM, 1), jnp.float32)
    for j in range(B // BN):
        Mt = jax.lax.dot_general(
            both,
            ut_ref[j * BN : (j + 1) * BN, :],
            (((1,), (1,)), ((), ())),
            preferred_element_type=jnp.float32,
        )
        St, QTt = Mt[:BM], Mt[BM:]
        Sm = jnp.where(cols + j * BN == rows + i * BM, -3.0, St)
        ml = jnp.max(Sm, axis=1, keepdims=True)
        pickl = jnp.sum(jnp.where(St == ml, QTt, 0.0), axis=1, keepdims=True)
        upd = ml > m
        m = jnp.where(upd, ml, m)
        pick = jnp.where(upd, pickl, pick)
    pick = pick[:, 0]
    diag = jnp.sum(uq * utb, axis=1)

    part = jnp.sum(jnp.maximum(1.0 - diag + pick, 0.0))
    out_ref[...] += part.reshape(1, 1) * (1.0 / B)


@jax.jit
def kernel(queries, targets):
    out = pl.pallas_call(
        _loss_kernel,
        grid=(GRID,),
        in_specs=[
            pl.BlockSpec((BM, D), lambda i: (i, 0)),
            pl.BlockSpec((B, D), lambda i: (0, 0)),
        ],
        out_specs=pl.BlockSpec((1, 1), lambda i: (0, 0)),
        out_shape=jax.ShapeDtypeStruct((1, 1), jnp.float32),
        scratch_shapes=[pltpu.VMEM((B, D), jnp.float32)],
    )(queries, targets)
    return out[0, 0]


# speedup vs baseline: 1.0642x; 1.0642x over previous
import functools

import jax
import jax.numpy as jnp
from jax.experimental import pallas as pl
import jax.experimental.pallas.tpu as pltpu

B = 4096
D = 128
BM = 4096
BN = 512
GRID = B // BM


def _loss_kernel(q_ref, t_ref, out_ref, ut_ref):
    i = pl.program_id(0)

    @pl.when(i == 0)
    def _init():
        t = t_ref[...]
        ut_ref[...] = t / jnp.sqrt(jnp.sum(t * t, axis=1, keepdims=True))
        out_ref[...] = jnp.zeros((1, 1), jnp.float32)

    utb = ut_ref[pl.ds(i * BM, BM), :]
    q = q_ref[...]
    uq = q / jnp.sqrt(jnp.sum(q * q, axis=1, keepdims=True))
    both = jnp.concatenate((utb, uq), axis=0)

    rows = jax.lax.broadcasted_iota(jnp.int32, (BM, BN), 0)
    cols = jax.lax.broadcasted_iota(jnp.int32, (BM, BN), 1)

    m = jnp.full((BM, 1), -4.0, jnp.float32)
    pick = jnp.zeros((BM, 1), jnp.float32)
    for j in range(B // BN):
        Mt = jax.lax.dot_general(
            both,
            ut_ref[j * BN : (j + 1) * BN, :],
            (((1,), (1,)), ((), ())),
            preferred_element_type=jnp.float32,
        )
        St, QTt = Mt[:BM], Mt[BM:]
        Sm = jnp.where(cols + j * BN == rows + i * BM, -3.0, St)
        ml = jnp.max(Sm, axis=1, keepdims=True)
        pickl = jnp.sum(jnp.where(St == ml, QTt, 0.0), axis=1, keepdims=True)
        upd = ml > m
        m = jnp.where(upd, ml, m)
        pick = jnp.where(upd, pickl, pick)
    pick = pick[:, 0]
    diag = jnp.sum(uq * utb, axis=1)

    part = jnp.sum(jnp.maximum(1.0 - diag + pick, 0.0))
    out_ref[...] += part.reshape(1, 1) * (1.0 / B)


@jax.jit
def kernel(queries, targets):
    out = pl.pallas_call(
        _loss_kernel,
        grid=(GRID,),
        in_specs=[
            pl.BlockSpec((BM, D), lambda i: (i, 0)),
            pl.BlockSpec((B, D), lambda i: (0, 0)),
        ],
        out_specs=pl.BlockSpec((1, 1), lambda i: (0, 0)),
        out_shape=jax.ShapeDtypeStruct((1, 1), jnp.float32),
        scratch_shapes=[pltpu.VMEM((B, D), jnp.float32)],
    )(queries, targets)
    return out[0, 0]


# speedup vs baseline: 1.1108x; 1.0438x over previous
import functools

import jax
import jax.numpy as jnp
from jax.experimental import pallas as pl
import jax.experimental.pallas.tpu as pltpu

B = 4096
D = 128
BM = 2048
BN = 512
GRID = B // BM


def _loss_kernel(q_ref, t_ref, out_ref, ut_ref):
    i = pl.program_id(0)

    @pl.when(i == 0)
    def _init():
        t = t_ref[...]
        ut_ref[...] = t / jnp.sqrt(jnp.sum(t * t, axis=1, keepdims=True))
        out_ref[...] = jnp.zeros((1, 1), jnp.float32)

    utb = ut_ref[pl.ds(i * BM, BM), :]
    q = q_ref[...]
    uq = q / jnp.sqrt(jnp.sum(q * q, axis=1, keepdims=True))
    both = jnp.concatenate((utb, uq), axis=0)

    rows = jax.lax.broadcasted_iota(jnp.int32, (BM, BN), 0)
    cols = jax.lax.broadcasted_iota(jnp.int32, (BM, BN), 1)

    m = jnp.full((BM, 1), -4.0, jnp.float32)
    pick = jnp.zeros((BM, 1), jnp.float32)
    for j in range(B // BN):
        Mt = jax.lax.dot_general(
            both,
            ut_ref[j * BN : (j + 1) * BN, :],
            (((1,), (1,)), ((), ())),
            preferred_element_type=jnp.float32,
        )
        St, QTt = Mt[:BM], Mt[BM:]
        Sm = jnp.where(cols + j * BN == rows + i * BM, -3.0, St)
        ml = jnp.max(Sm, axis=1, keepdims=True)
        pickl = jnp.sum(jnp.where(St == ml, QTt, 0.0), axis=1, keepdims=True)
        upd = ml > m
        m = jnp.where(upd, ml, m)
        pick = jnp.where(upd, pickl, pick)
    pick = pick[:, 0]
    diag = jnp.sum(uq * utb, axis=1)

    part = jnp.sum(jnp.maximum(1.0 - diag + pick, 0.0))
    out_ref[...] += part.reshape(1, 1) * (1.0 / B)


@jax.jit
def kernel(queries, targets):
    out = pl.pallas_call(
        _loss_kernel,
        grid=(GRID,),
        in_specs=[
            pl.BlockSpec((BM, D), lambda i: (i, 0)),
            pl.BlockSpec((B, D), lambda i: (0, 0)),
        ],
        out_specs=pl.BlockSpec((1, 1), lambda i: (0, 0)),
        out_shape=jax.ShapeDtypeStruct((1, 1), jnp.float32),
        scratch_shapes=[pltpu.VMEM((B, D), jnp.float32)],
    )(queries, targets)
    return out[0, 0]
